# baseline (speedup 1.0000x reference)
import functools

import jax
import jax.numpy as jnp
from jax import lax
from jax.experimental import pallas as pl
from jax.experimental.pallas import tpu as pltpu
from jax.experimental.pallas import tpu_sc as plsc

DIM = 64
B = 16384
LAMBD = 0.1

NC = 2
NS = 16
L = 16
NW = NC * NS
BPW = B // NW
CHUNK = 128
NCHUNK = BPW // CHUNK
GROUPS = CHUNK // L


def _sc_gather_reduce(sources, contexts, pure_sources, personas,
                      node_t, noise_t, base_t):
  mesh = plsc.VectorSubcoreMesh(core_axis_name="c", subcore_axis_name="s")
  out_type = [jax.ShapeDtypeStruct((B,), jnp.float32)] * 4
  scratch = [
      pltpu.VMEM((2 * NCHUNK, CHUNK), jnp.int32),
      pltpu.VMEM((2 * NCHUNK, CHUNK), jnp.int32),
      pltpu.VMEM((DIM, CHUNK), jnp.float32),
      pltpu.VMEM((DIM, CHUNK), jnp.float32),
      pltpu.VMEM((BPW,), jnp.float32),
      pltpu.VMEM((BPW,), jnp.float32),
      pltpu.VMEM((BPW,), jnp.float32),
      pltpu.VMEM((BPW,), jnp.float32),
      pltpu.SemaphoreType.DMA,
      pltpu.SemaphoreType.DMA,
  ]

  @functools.partial(
      pl.kernel, mesh=mesh, out_type=out_type, scratch_types=scratch,
      compiler_params=pltpu.CompilerParams(
          needs_layout_passes=False, use_tc_tiling_on_sc=False))
  def k(src_hbm, ctx_hbm, psrc_hbm, pers_hbm,
        node_hbm, noise_hbm, base_hbm,
        s1_hbm, n1_hbm, n2_hbm, r_hbm,
        idx_a, idx_b, rows_a, rows_b,
        s1_v, n1_v, n2_v, r_v, sem_i, sem_g):
    wid = lax.axis_index("s") * NC + lax.axis_index("c")
    base = wid * BPW

    phases = [(src_hbm, ctx_hbm, node_hbm, noise_hbm, True),
              (psrc_hbm, pers_hbm, node_hbm, base_hbm, False)]

    idx_copies = []
    for ph, (ia_h, ib_h, _, _, _) in enumerate(phases):
      for c in range(NCHUNK):
        off = base + c * CHUNK
        s = ph * NCHUNK + c
        idx_copies.append(
            pltpu.async_copy(ia_h.at[pl.ds(off, CHUNK)], idx_a.at[s], sem_i))
        idx_copies.append(
            pltpu.async_copy(ib_h.at[pl.ds(off, CHUNK)], idx_b.at[s], sem_i))
    for cp in idx_copies:
      cp.wait()

    for ph, (_, _, ta, tb, pair1) in enumerate(phases):
      for c in range(NCHUNK):
        s = ph * NCHUNK + c

        def fire(j, _, ta=ta, tb=tb, s=s):
          pltpu.async_copy(ta.at[j].at[idx_a.at[s]], rows_a.at[j], sem_g)
          pltpu.async_copy(tb.at[j].at[idx_b.at[s]], rows_b.at[j], sem_g)
          return 0

        lax.fori_loop(0, DIM, fire, 0)

        def drain(j, _, ta=ta, tb=tb, s=s):
          pltpu.make_async_copy(
              ta.at[0].at[idx_a.at[s]], rows_a.at[0], sem_g).wait()
          pltpu.make_async_copy(
              tb.at[0].at[idx_b.at[s]], rows_b.at[0], sem_g).wait()
          return 0

        lax.fori_loop(0, DIM, drain, 0)

        def group(g, _, c=c, pair1=pair1):
          sl = pl.ds(g * L, L)
          s1 = n1 = n2 = jnp.zeros((L,), jnp.float32)
          for j in range(DIM):
            av = rows_a[j, sl]
            bv = rows_b[j, sl]
            s1 = s1 + av * bv
            if pair1:
              n1 = n1 + av * av
              n2 = n2 + bv * bv
          o = c * CHUNK + g * L
          if pair1:
            s1_v[pl.ds(o, L)] = s1
            n1_v[pl.ds(o, L)] = n1
            n2_v[pl.ds(o, L)] = n2
          else:
            r_v[pl.ds(o, L)] = s1
          return 0

        lax.fori_loop(0, GROUPS, group, 0)

    pltpu.sync_copy(s1_v, s1_hbm.at[pl.ds(base, BPW)])
    pltpu.sync_copy(n1_v, n1_hbm.at[pl.ds(base, BPW)])
    pltpu.sync_copy(n2_v, n2_hbm.at[pl.ds(base, BPW)])
    pltpu.sync_copy(r_v, r_hbm.at[pl.ds(base, BPW)])

  return k(sources, contexts, pure_sources, personas,
           node_t, noise_t, base_t)


def _tc_loss(s1, n1, n2, r, targets):
  def body(s1_ref, n1_ref, n2_ref, r_ref, t_ref, out_ref):
    s1v = s1_ref[...]
    na = jnp.maximum(jnp.sqrt(n1_ref[...]), 1e-12)
    nb = jnp.maximum(jnp.sqrt(n2_ref[...]), 1e-12)
    t = t_ref[...]
    score = jax.nn.sigmoid(s1v / (na * nb))
    main = t * jnp.log(score) + (1.0 - t) * jnp.log(1.0 - score)
    main_loss = -jnp.sum(main) / B
    rs = jax.nn.sigmoid(jnp.clip(r_ref[...], -15.0, 15.0))
    reg_loss = -jnp.sum(jnp.log(rs)) / B
    out_ref[0, 0] = main_loss + LAMBD * reg_loss

  side = 128
  return pl.pallas_call(
      body,
      out_shape=jax.ShapeDtypeStruct((1, 1), jnp.float32),
      out_specs=pl.BlockSpec(memory_space=pltpu.SMEM),
  )(s1.reshape(side, side), n1.reshape(side, side), n2.reshape(side, side),
    r.reshape(side, side), targets.reshape(side, side))


def kernel(sources, contexts, targets, personas, pure_sources,
           node_embedding, node_noise_embedding, base_node_embedding):
  s1, n1, n2, r = _sc_gather_reduce(
      sources.astype(jnp.int32), contexts.astype(jnp.int32),
      pure_sources.astype(jnp.int32), personas.astype(jnp.int32),
      node_embedding.T, node_noise_embedding.T, base_node_embedding.T)
  loss = _tc_loss(s1, n1, n2, r, targets)
  return loss[0, 0]
